# baseline (speedup 1.0000x reference)
import functools

import jax
import jax.numpy as jnp
from jax.experimental import pallas as pl
from jax.experimental.pallas import tpu as pltpu

_P = 4


def _blockdiag4(w):
    z = jnp.zeros_like(w)
    rows = [jnp.concatenate([w if j == i else z for j in range(_P)], axis=1)
            for i in range(_P)]
    return jnp.concatenate(rows, axis=0)


def _cell_body(enc_ref, mask_ref, keys_ref, u_ref, v_ref, w_ref,
               out_ref, xw_ref, xc_ref, *, S, KQ, bt, D):
    DP = _P * D
    li = jax.lax.broadcasted_iota(jnp.int32, (DP, DP), 0)
    lj = jax.lax.broadcasted_iota(jnp.int32, (DP, DP), 1)
    A = (li // D == lj // D).astype(jnp.float32)
    U4 = _blockdiag4(u_ref[...])
    V4 = _blockdiag4(v_ref[...])
    W4 = _blockdiag4(w_ref[...])

    xc_ref[...] = jnp.concatenate([enc_ref[...]] * _P, axis=2)
    xw_ref[...] = jnp.dot(xc_ref[...].reshape(S * bt, DP), W4,
                          preferred_element_type=jnp.float32).reshape(S, bt, DP)
    keys3 = keys_ref[...]

    KA = KQ // 2
    keys_h = (keys3[:KA], keys3[KA:])
    kV_h = tuple(
        jnp.dot(k3.reshape(-1, DP), V4,
                preferred_element_type=jnp.float32).reshape(k3.shape)
        for k3 in keys_h)

    def half_step(h3, k3, kV3, xc, xw, m):
        kq = h3.shape[0]
        t = xc[None, :, :] * (h3 + k3)
        g = jax.nn.sigmoid(
            jnp.dot(t.reshape(kq * bt, DP), A,
                    preferred_element_type=jnp.float32).reshape(kq, bt, DP))
        hU = jnp.dot(h3.reshape(kq * bt, DP), U4,
                     preferred_element_type=jnp.float32).reshape(kq, bt, DP)
        ht = jnp.maximum(hU + kV3 + xw[None, :, :], 0.0)
        upd = h3 + g * ht
        ss = jnp.dot((upd * upd).reshape(kq * bt, DP), A,
                     preferred_element_type=jnp.float32).reshape(kq, bt, DP)
        upd = upd * jax.lax.rsqrt(jnp.maximum(ss, 1e-12))
        return jnp.where(m[None, :, :] != 0.0, upd, h3)

    def step(i, hs):
        xc = xc_ref[i]
        xw = xw_ref[i]
        m = mask_ref[i]
        return tuple(
            half_step(h3, k3, kv3, xc, xw, m)
            for h3, k3, kv3 in zip(hs, keys_h, kV_h))

    h0 = (jnp.zeros((KA, bt, DP), jnp.float32),
          jnp.zeros((KQ - KA, bt, DP), jnp.float32))
    hA, hB = jax.lax.fori_loop(0, S, step, h0, unroll=S)
    out_ref[:KA] = hA
    out_ref[KA:] = hB


def kernel(encoded_sents, mask, keys, U, V, W):
    B, S, D = encoded_sents.shape
    K = keys.shape[1]
    KQ = K // _P
    DP = _P * D
    bt = 256

    enc_t = jnp.swapaxes(encoded_sents, 0, 1)
    mask_t = jnp.swapaxes(mask, 0, 1).astype(jnp.float32)[:, :, None]
    keys_p = (keys.reshape(B, KQ, _P, D)
              .transpose(1, 0, 2, 3)
              .reshape(KQ, B, DP))

    out = pl.pallas_call(
        functools.partial(_cell_body, S=S, KQ=KQ, bt=bt, D=D),
        grid=(B // bt,),
        in_specs=[
            pl.BlockSpec((S, bt, D), lambda i: (0, i, 0)),
            pl.BlockSpec((S, bt, 1), lambda i: (0, i, 0)),
            pl.BlockSpec((KQ, bt, DP), lambda i: (0, i, 0)),
            pl.BlockSpec((D, D), lambda i: (0, 0)),
            pl.BlockSpec((D, D), lambda i: (0, 0)),
            pl.BlockSpec((D, D), lambda i: (0, 0)),
        ],
        out_specs=pl.BlockSpec((KQ, bt, DP), lambda i: (0, i, 0)),
        out_shape=jax.ShapeDtypeStruct((KQ, B, DP), jnp.float32),
        scratch_shapes=[pltpu.VMEM((S, bt, DP), jnp.float32),
                        pltpu.VMEM((S, bt, DP), jnp.float32)],
        compiler_params=pltpu.CompilerParams(
            dimension_semantics=("parallel",)),
    )(enc_t, mask_t, keys_p, U, V, W)

    return (out.reshape(KQ, B, _P, D)
            .transpose(1, 0, 2, 3)
            .reshape(B, K, D))


# speedup vs baseline: 1.3613x; 1.3613x over previous
import functools

import jax
import jax.numpy as jnp
from jax.experimental import pallas as pl
from jax.experimental.pallas import tpu as pltpu

_P = 4


def _blockdiag4(w):
    z = jnp.zeros_like(w)
    rows = [jnp.concatenate([w if j == i else z for j in range(_P)], axis=1)
            for i in range(_P)]
    return jnp.concatenate(rows, axis=0)


def _cell_body(enc_ref, mask_ref, keys_ref, u_ref, v_ref, w_ref,
               out_ref, xw_ref, xc_ref, *, S, KQ, bt, D):
    DP = _P * D
    li = jax.lax.broadcasted_iota(jnp.int32, (DP, DP), 0)
    lj = jax.lax.broadcasted_iota(jnp.int32, (DP, DP), 1)
    A = (li // D == lj // D).astype(jnp.float32)
    U4 = _blockdiag4(u_ref[...])
    V4 = _blockdiag4(v_ref[...])
    W4 = _blockdiag4(w_ref[...])

    xc_ref[...] = jnp.concatenate([enc_ref[...]] * _P, axis=2)
    xw_ref[...] = jnp.dot(xc_ref[...].reshape(S * bt, DP), W4,
                          preferred_element_type=jnp.float32).reshape(S, bt, DP)
    keys3 = keys_ref[...]

    KA = KQ
    keys_h = (keys3,)
    kV_h = tuple(
        jnp.dot(k3.reshape(-1, DP), V4,
                preferred_element_type=jnp.float32).reshape(k3.shape)
        for k3 in keys_h)

    def half_step(h3, k3, kV3, xc, xw, m):
        kq = h3.shape[0]
        t = xc[None, :, :] * (h3 + k3)
        g = jax.nn.sigmoid(
            jnp.dot(t.reshape(kq * bt, DP), A,
                    preferred_element_type=jnp.float32).reshape(kq, bt, DP))
        hU = jnp.dot(h3.reshape(kq * bt, DP), U4,
                     preferred_element_type=jnp.float32).reshape(kq, bt, DP)
        ht = jnp.maximum(hU + kV3 + xw[None, :, :], 0.0)
        upd = h3 + g * ht
        ss = jnp.dot((upd * upd).reshape(kq * bt, DP), A,
                     preferred_element_type=jnp.float32).reshape(kq, bt, DP)
        upd = upd * jax.lax.rsqrt(jnp.maximum(ss, 1e-12))
        return jnp.where(m[None, :, :] != 0.0, upd, h3)

    def step(i, hs):
        xc = xc_ref[i]
        xw = xw_ref[i]
        m = mask_ref[i]
        return tuple(
            half_step(h3, k3, kv3, xc, xw, m)
            for h3, k3, kv3 in zip(hs, keys_h, kV_h))

    h0 = (jnp.zeros((KQ, bt, DP), jnp.float32),)
    hs = jax.lax.fori_loop(0, S, step, h0, unroll=S)
    out_ref[...] = hs[0]


def kernel(encoded_sents, mask, keys, U, V, W):
    B, S, D = encoded_sents.shape
    K = keys.shape[1]
    KQ = K // _P
    DP = _P * D
    bt = 128

    enc_t = jnp.swapaxes(encoded_sents, 0, 1)
    mask_t = jnp.swapaxes(mask, 0, 1).astype(jnp.float32)[:, :, None]
    keys_p = (keys.reshape(B, KQ, _P, D)
              .transpose(1, 0, 2, 3)
              .reshape(KQ, B, DP))

    out = pl.pallas_call(
        functools.partial(_cell_body, S=S, KQ=KQ, bt=bt, D=D),
        grid=(B // bt,),
        in_specs=[
            pl.BlockSpec((S, bt, D), lambda i: (0, i, 0)),
            pl.BlockSpec((S, bt, 1), lambda i: (0, i, 0)),
            pl.BlockSpec((KQ, bt, DP), lambda i: (0, i, 0)),
            pl.BlockSpec((D, D), lambda i: (0, 0)),
            pl.BlockSpec((D, D), lambda i: (0, 0)),
            pl.BlockSpec((D, D), lambda i: (0, 0)),
        ],
        out_specs=pl.BlockSpec((KQ, bt, DP), lambda i: (0, i, 0)),
        out_shape=jax.ShapeDtypeStruct((KQ, B, DP), jnp.float32),
        scratch_shapes=[pltpu.VMEM((S, bt, DP), jnp.float32),
                        pltpu.VMEM((S, bt, DP), jnp.float32)],
        compiler_params=pltpu.CompilerParams(
            dimension_semantics=("parallel",)),
    )(enc_t, mask_t, keys_p, U, V, W)

    return (out.reshape(KQ, B, _P, D)
            .transpose(1, 0, 2, 3)
            .reshape(B, K, D))
